# baseline (speedup 1.0000x reference)
import math

import jax
import jax.numpy as jnp
from jax import lax
from jax.experimental import pallas as pl
from jax.experimental.pallas import tpu as pltpu
from jax.experimental.pallas import tpu_sc as plsc

N = 10000
E = 320000
IN_DIM = 128
H = 8
D = 16
EPS = 1e-09
SCALE = 1.0 / math.sqrt(D)

NC = 2
NS = 16
NW = NC * NS
EPW = E // NW
CH = 40
NCHUNK = EPW // CH
WB = 80
NWB = N // WB
WB_PER_TILE = -(-NWB // NS)

_PERM = []
for _j in range(H // 2):
    for _i in range(D):
        _PERM.extend((32 * _j + _i, 32 * _j + 16 + _i))
_PERM = tuple(_PERM)


def _proj_body(x_ref, wq_ref, bq_ref, wk_ref, bk_ref, wv_ref, bv_ref,
               q_ref, k_ref, v_ref):
    xb = x_ref[...]
    q_ref[...] = (jnp.dot(xb, wq_ref[...], preferred_element_type=jnp.float32)
                  + bq_ref[...]).astype(jnp.bfloat16)
    k_ref[...] = (jnp.dot(xb, wk_ref[...], preferred_element_type=jnp.float32)
                  + bk_ref[...]).astype(jnp.bfloat16)
    v_ref[...] = jnp.dot(xb, wv_ref[...],
                         preferred_element_type=jnp.float32) + bv_ref[...]


def _node_proj(x, WQ, bQ, WK, bK, WV, bV):
    blk = 1000
    grid = N // blk
    wspec = pl.BlockSpec((IN_DIM, H * D), lambda i: (0, 0))
    bspec = pl.BlockSpec((1, H * D), lambda i: (0, 0))
    ospec = pl.BlockSpec((blk, H * D), lambda i: (i, 0))
    return pl.pallas_call(
        _proj_body,
        grid=(grid,),
        in_specs=[pl.BlockSpec((blk, IN_DIM), lambda i: (i, 0)),
                  wspec, bspec, wspec, bspec, wspec, bspec],
        out_specs=[ospec, ospec, ospec],
        out_shape=[jax.ShapeDtypeStruct((N, H * D), jnp.bfloat16),
                   jax.ShapeDtypeStruct((N, H * D), jnp.bfloat16),
                   jax.ShapeDtypeStruct((N, H * D), jnp.float32)],
    )(x, WQ, bQ.reshape(1, -1), WK, bK.reshape(1, -1), WV, bV.reshape(1, -1))


def _edge_proj_body(ea_ref, we_ref, be_ref, eh_ref):
    eh_ref[...] = jnp.dot(ea_ref[...], we_ref[...],
                          preferred_element_type=jnp.float32) + be_ref[...]


def _edge_proj(edge_attr, WE, bE):
    blk = 4000
    grid = E // blk
    return pl.pallas_call(
        _edge_proj_body,
        grid=(grid,),
        in_specs=[pl.BlockSpec((blk, IN_DIM), lambda i: (i, 0)),
                  pl.BlockSpec((IN_DIM, H * D), lambda i: (0, 0)),
                  pl.BlockSpec((1, H * D), lambda i: (0, 0))],
        out_specs=pl.BlockSpec((blk, H * D), lambda i: (i, 0)),
        out_shape=jax.ShapeDtypeStruct((E, H * D), jnp.float32),
    )(edge_attr, WE, bE.reshape(1, -1))


def _edge_kernel(src_hbm, dst_hbm, qh_hbm, kh_hbm, vh_hbm, eh_hbm,
                 owv_hbm, owz_hbm,
                 src_v, dst_v, k_v, q_v, v_v, e_v, s_v, wb_v, wbz_v,
                 sem, wv_sh, wz_sh):
    cid = lax.axis_index("c")
    sid = lax.axis_index("s")
    wid = sid * NC + cid

    zero16 = jnp.zeros((16,), jnp.float32)

    def _zrow(r, carry):
        for cc in range(8):
            wb_v[r, pl.ds(cc * 16, 16)] = zero16
        wbz_v[r, :] = zero16
        return carry
    lax.fori_loop(0, WB, _zrow, 0)

    for j in range(WB_PER_TILE):
        ci = sid + j * NS
        @pl.when(ci < NWB)
        def _init():
            r0 = ci * WB
            pltpu.sync_copy(wb_v, wv_sh.at[pl.ds(r0, WB)])
            pltpu.sync_copy(wbz_v, wz_sh.at[pl.ds(r0, WB)])
    plsc.subcore_barrier()

    lane = lax.iota(jnp.int32, 16)
    onehot = [(lane == h).astype(jnp.float32) for h in range(H)]
    perms = [lane ^ st for st in (8, 4, 2, 1)]
    bidx = [lane * 0 + h for h in range(H)]

    ebase = wid * EPW

    def _chunk(c, carry):
        e0 = ebase + c * CH
        pltpu.sync_copy(src_hbm.at[pl.ds(e0, CH)], src_v)
        pltpu.sync_copy(dst_hbm.at[pl.ds(e0, CH)], dst_v)
        cp_k = pltpu.async_copy(kh_hbm.at[src_v], k_v, sem)
        cp_q = pltpu.async_copy(qh_hbm.at[dst_v], q_v, sem)
        cp_v = pltpu.async_copy(vh_hbm.at[src_v], v_v, sem)
        pltpu.sync_copy(eh_hbm.at[pl.ds(e0, CH)], e_v)
        cp_k.wait()
        cp_q.wait()
        cp_v.wait()

        def _edge(e, ecarry):
            p = []
            for j in range(H // 2):
                ka, kb2 = plsc.unpack(k_v[e, pl.ds(32 * j, 32)],
                                      format=plsc.PackFormat.INTERLEAVED,
                                      preferred_element_type=jnp.float32)
                qa, qb2 = plsc.unpack(q_v[e, pl.ds(32 * j, 32)],
                                      format=plsc.PackFormat.INTERLEAVED,
                                      preferred_element_type=jnp.float32)
                p.append(ka * qa * e_v[e, pl.ds(32 * j, 16)])
                p.append(kb2 * qb2 * e_v[e, pl.ds(32 * j + 16, 16)])
            for pm in perms:
                p = [ph + jnp.take(ph, pm) for ph in p]
            srow = p[0] * onehot[0]
            for h in range(1, H):
                srow = srow + p[h] * onehot[h]
            svec_all = jnp.exp(srow * SCALE)
            s_v[e, :] = svec_all
            for h in range(H):
                sv = jnp.take(svec_all, bidx[h])
                v_v[e, pl.ds(h * 16, 16)] = v_v[e, pl.ds(h * 16, 16)] * sv
            return ecarry
        lax.fori_loop(0, CH, _edge, 0)

        pltpu.sync_copy(v_v, wv_sh.at[dst_v], add=True)
        pltpu.sync_copy(s_v, wz_sh.at[dst_v], add=True)
        return carry
    lax.fori_loop(0, NCHUNK, _chunk, 0)

    plsc.subcore_barrier()

    for j in range(WB_PER_TILE):
        ci = sid + j * NS
        @pl.when(ci < NWB)
        def _wb():
            r0 = ci * WB
            pltpu.sync_copy(wv_sh.at[pl.ds(r0, WB)], wb_v)
            pltpu.sync_copy(wb_v, owv_hbm.at[pl.ds(cid * N + r0, WB)])
            pltpu.sync_copy(wz_sh.at[pl.ds(r0, WB)], wbz_v)
            pltpu.sync_copy(wbz_v, owz_hbm.at[pl.ds(cid * N + r0, WB)])


def _edge_stage(src, dst, Qh, Kh, Vh, Eh):
    mesh = plsc.VectorSubcoreMesh(core_axis_name="c", subcore_axis_name="s")
    f = pl.kernel(
        _edge_kernel,
        out_type=[jax.ShapeDtypeStruct((NC * N, H * D), jnp.float32),
                  jax.ShapeDtypeStruct((NC * N, D), jnp.float32)],
        mesh=mesh,
        compiler_params=pltpu.CompilerParams(needs_layout_passes=False,
                                             use_tc_tiling_on_sc=False),
        scratch_types=[
            pltpu.VMEM((CH,), jnp.int32),
            pltpu.VMEM((CH,), jnp.int32),
            pltpu.VMEM((CH, H * D), jnp.bfloat16),
            pltpu.VMEM((CH, H * D), jnp.bfloat16),
            pltpu.VMEM((CH, H * D), jnp.float32),
            pltpu.VMEM((CH, H * D), jnp.float32),
            pltpu.VMEM((CH, D), jnp.float32),
            pltpu.VMEM((WB, H * D), jnp.float32),
            pltpu.VMEM((WB, D), jnp.float32),
            pltpu.SemaphoreType.DMA,
            pltpu.VMEM_SHARED((N, H * D), jnp.float32),
            pltpu.VMEM_SHARED((N, D), jnp.float32),
        ],
    )
    return f(src, dst, Qh, Kh, Vh, Eh)


def _final_body(wv_ref, wz_ref, out_ref):
    wv = wv_ref[0] + wv_ref[1]
    wz = wz_ref[0] + wz_ref[1]
    for h in range(H):
        denom = wz[:, h:h + 1] + EPS
        out_ref[:, h * D:(h + 1) * D] = wv[:, h * D:(h + 1) * D] / denom


def _finalize(owv, owz):
    blk = 1000
    grid = N // blk
    wv2 = owv.reshape(NC, N, H * D)
    wz2 = owz.reshape(NC, N, D)
    return pl.pallas_call(
        _final_body,
        grid=(grid,),
        in_specs=[pl.BlockSpec((NC, blk, H * D), lambda i: (0, i, 0)),
                  pl.BlockSpec((NC, blk, D), lambda i: (0, i, 0))],
        out_specs=pl.BlockSpec((blk, H * D), lambda i: (i, 0)),
        out_shape=jax.ShapeDtypeStruct((N, H * D), jnp.float32),
    )(wv2, wz2)


def kernel(x, edge_attr, edge_index, WQ, bQ, WK, bK, WV, bV, WE, bE):
    perm = jnp.array(_PERM, dtype=jnp.int32)
    Qh, Kh, Vh = _node_proj(x, WQ[:, perm], bQ[perm], WK[:, perm], bK[perm],
                            WV, bV)
    Eh = _edge_proj(edge_attr, WE, bE)
    src = edge_index[0]
    dst = edge_index[1]
    owv, owz = _edge_stage(src, dst, Qh, Kh, Vh, Eh)
    out = _finalize(owv, owz)
    return out.reshape(N, H, D)


# speedup vs baseline: 1.1535x; 1.1535x over previous
import math

import jax
import jax.numpy as jnp
from jax import lax
from jax.experimental import pallas as pl
from jax.experimental.pallas import tpu as pltpu
from jax.experimental.pallas import tpu_sc as plsc

N = 10000
E = 320000
IN_DIM = 128
H = 8
D = 16
EPS = 1e-09
SCALE = 1.0 / math.sqrt(D)

NC = 2
NS = 16
NW = NC * NS
EPW = E // NW
CH = 80
NCHUNK = EPW // CH
WB = 80
NWB = N // WB
WB_PER_TILE = -(-NWB // NS)

_PERM = []
for _j in range(H // 2):
    for _i in range(D):
        _PERM.extend((32 * _j + _i, 32 * _j + 16 + _i))
_PERM = tuple(_PERM)


def _proj_body(x_ref, wq_ref, bq_ref, wk_ref, bk_ref, wv_ref, bv_ref,
               q_ref, k_ref, v_ref):
    xb = x_ref[...]
    q_ref[...] = (jnp.dot(xb, wq_ref[...], preferred_element_type=jnp.float32)
                  + bq_ref[...]).astype(jnp.bfloat16)
    k_ref[...] = (jnp.dot(xb, wk_ref[...], preferred_element_type=jnp.float32)
                  + bk_ref[...]).astype(jnp.bfloat16)
    v_ref[...] = jnp.dot(xb, wv_ref[...],
                         preferred_element_type=jnp.float32) + bv_ref[...]


def _node_proj(x, WQ, bQ, WK, bK, WV, bV):
    blk = 1000
    grid = N // blk
    wspec = pl.BlockSpec((IN_DIM, H * D), lambda i: (0, 0))
    bspec = pl.BlockSpec((1, H * D), lambda i: (0, 0))
    ospec = pl.BlockSpec((blk, H * D), lambda i: (i, 0))
    return pl.pallas_call(
        _proj_body,
        grid=(grid,),
        in_specs=[pl.BlockSpec((blk, IN_DIM), lambda i: (i, 0)),
                  wspec, bspec, wspec, bspec, wspec, bspec],
        out_specs=[ospec, ospec, ospec],
        out_shape=[jax.ShapeDtypeStruct((N, H * D), jnp.bfloat16),
                   jax.ShapeDtypeStruct((N, H * D), jnp.bfloat16),
                   jax.ShapeDtypeStruct((N, H * D), jnp.float32)],
    )(x, WQ, bQ.reshape(1, -1), WK, bK.reshape(1, -1), WV, bV.reshape(1, -1))


def _edge_proj_body(ea_ref, we_ref, be_ref, eh_ref):
    eh_ref[...] = jnp.dot(ea_ref[...], we_ref[...],
                          preferred_element_type=jnp.float32) + be_ref[...]


def _edge_proj(edge_attr, WE, bE):
    blk = 4000
    grid = E // blk
    return pl.pallas_call(
        _edge_proj_body,
        grid=(grid,),
        in_specs=[pl.BlockSpec((blk, IN_DIM), lambda i: (i, 0)),
                  pl.BlockSpec((IN_DIM, H * D), lambda i: (0, 0)),
                  pl.BlockSpec((1, H * D), lambda i: (0, 0))],
        out_specs=pl.BlockSpec((blk, H * D), lambda i: (i, 0)),
        out_shape=jax.ShapeDtypeStruct((E, H * D), jnp.float32),
    )(edge_attr, WE, bE.reshape(1, -1))


def _edge_kernel(src_hbm, dst_hbm, qh_hbm, kh_hbm, vh_hbm, eh_hbm,
                 owv_hbm, owz_hbm,
                 src_v, dst_v, k_v, q_v, v_v, e_v, s_v,
                 sem, wv_sh, wz_sh):
    cid = lax.axis_index("c")
    sid = lax.axis_index("s")
    wid = sid * NC + cid

    zero16 = jnp.zeros((16,), jnp.float32)

    def _zrow(r, carry):
        for cc in range(8):
            e_v[r, pl.ds(cc * 16, 16)] = zero16
        s_v[r, :] = zero16
        return carry
    lax.fori_loop(0, WB, _zrow, 0)

    for j in range(WB_PER_TILE):
        ci = sid + j * NS
        @pl.when(ci < NWB)
        def _init():
            r0 = ci * WB
            pltpu.sync_copy(e_v, wv_sh.at[pl.ds(r0, WB)])
            pltpu.sync_copy(s_v, wz_sh.at[pl.ds(r0, WB)])
    plsc.subcore_barrier()

    lane = lax.iota(jnp.int32, 16)
    onehot = [(lane == h).astype(jnp.float32) for h in range(H)]
    perms = [lane ^ st for st in (8, 4, 2, 1)]
    bidx = [lane * 0 + h for h in range(H)]

    ebase = wid * EPW

    def _chunk(c, carry):
        e0 = ebase + c * CH
        pltpu.sync_copy(src_hbm.at[pl.ds(e0, CH)], src_v)
        pltpu.sync_copy(dst_hbm.at[pl.ds(e0, CH)], dst_v)
        cp_k = pltpu.async_copy(kh_hbm.at[src_v], k_v, sem)
        cp_q = pltpu.async_copy(qh_hbm.at[dst_v], q_v, sem)
        cp_v = pltpu.async_copy(vh_hbm.at[src_v], v_v, sem)
        pltpu.sync_copy(eh_hbm.at[pl.ds(e0, CH)], e_v)
        cp_k.wait()
        cp_q.wait()
        cp_v.wait()

        def _edge(e, ecarry):
            p = []
            for j in range(H // 2):
                ka, kb2 = plsc.unpack(k_v[e, pl.ds(32 * j, 32)],
                                      format=plsc.PackFormat.INTERLEAVED,
                                      preferred_element_type=jnp.float32)
                qa, qb2 = plsc.unpack(q_v[e, pl.ds(32 * j, 32)],
                                      format=plsc.PackFormat.INTERLEAVED,
                                      preferred_element_type=jnp.float32)
                p.append(ka * qa * e_v[e, pl.ds(32 * j, 16)])
                p.append(kb2 * qb2 * e_v[e, pl.ds(32 * j + 16, 16)])
            for pm in perms:
                p = [ph + jnp.take(ph, pm) for ph in p]
            srow = p[0] * onehot[0]
            for h in range(1, H):
                srow = srow + p[h] * onehot[h]
            svec_all = jnp.exp(srow * SCALE)
            s_v[e, :] = svec_all
            for h in range(H):
                sv = jnp.take(svec_all, bidx[h])
                v_v[e, pl.ds(h * 16, 16)] = v_v[e, pl.ds(h * 16, 16)] * sv
            return ecarry
        lax.fori_loop(0, CH, _edge, 0)

        pltpu.sync_copy(v_v, wv_sh.at[dst_v], add=True)
        pltpu.sync_copy(s_v, wz_sh.at[dst_v], add=True)
        return carry
    lax.fori_loop(0, NCHUNK, _chunk, 0)

    plsc.subcore_barrier()

    for j in range(WB_PER_TILE):
        ci = sid + j * NS
        @pl.when(ci < NWB)
        def _wb():
            r0 = ci * WB
            pltpu.sync_copy(wv_sh.at[pl.ds(r0, WB)], e_v)
            pltpu.sync_copy(e_v, owv_hbm.at[pl.ds(cid * N + r0, WB)])
            pltpu.sync_copy(wz_sh.at[pl.ds(r0, WB)], s_v)
            pltpu.sync_copy(s_v, owz_hbm.at[pl.ds(cid * N + r0, WB)])


def _edge_stage(src, dst, Qh, Kh, Vh, Eh):
    mesh = plsc.VectorSubcoreMesh(core_axis_name="c", subcore_axis_name="s")
    f = pl.kernel(
        _edge_kernel,
        out_type=[jax.ShapeDtypeStruct((NC * N, H * D), jnp.float32),
                  jax.ShapeDtypeStruct((NC * N, D), jnp.float32)],
        mesh=mesh,
        compiler_params=pltpu.CompilerParams(needs_layout_passes=False,
                                             use_tc_tiling_on_sc=False),
        scratch_types=[
            pltpu.VMEM((CH,), jnp.int32),
            pltpu.VMEM((CH,), jnp.int32),
            pltpu.VMEM((CH, H * D), jnp.bfloat16),
            pltpu.VMEM((CH, H * D), jnp.bfloat16),
            pltpu.VMEM((CH, H * D), jnp.float32),
            pltpu.VMEM((CH, H * D), jnp.float32),
            pltpu.VMEM((CH, D), jnp.float32),
            pltpu.SemaphoreType.DMA,
            pltpu.VMEM_SHARED((N, H * D), jnp.float32),
            pltpu.VMEM_SHARED((N, D), jnp.float32),
        ],
    )
    return f(src, dst, Qh, Kh, Vh, Eh)


def _final_body(wv_ref, wz_ref, out_ref):
    wv = wv_ref[0] + wv_ref[1]
    wz = wz_ref[0] + wz_ref[1]
    for h in range(H):
        denom = wz[:, h:h + 1] + EPS
        out_ref[:, h * D:(h + 1) * D] = wv[:, h * D:(h + 1) * D] / denom


def _finalize(owv, owz):
    blk = 1000
    grid = N // blk
    wv2 = owv.reshape(NC, N, H * D)
    wz2 = owz.reshape(NC, N, D)
    return pl.pallas_call(
        _final_body,
        grid=(grid,),
        in_specs=[pl.BlockSpec((NC, blk, H * D), lambda i: (0, i, 0)),
                  pl.BlockSpec((NC, blk, D), lambda i: (0, i, 0))],
        out_specs=pl.BlockSpec((blk, H * D), lambda i: (i, 0)),
        out_shape=jax.ShapeDtypeStruct((N, H * D), jnp.float32),
    )(wv2, wz2)


def kernel(x, edge_attr, edge_index, WQ, bQ, WK, bK, WV, bV, WE, bE):
    perm = jnp.array(_PERM, dtype=jnp.int32)
    Qh, Kh, Vh = _node_proj(x, WQ[:, perm], bQ[perm], WK[:, perm], bK[perm],
                            WV, bV)
    Eh = _edge_proj(edge_attr, WE, bE)
    src = edge_index[0]
    dst = edge_index[1]
    owv, owz = _edge_stage(src, dst, Qh, Kh, Vh, Eh)
    out = _finalize(owv, owz)
    return out.reshape(N, H, D)
